# Initial kernel scaffold; baseline (speedup 1.0000x reference)
import jax
import jax.numpy as jnp
from jax.experimental import pallas as pl


def kernel(inputs, embeddings):
    raise NotImplementedError("write your pallas kernel here")


# speedup vs baseline: 1.4852x; 1.4852x over previous
import functools

import jax
import jax.numpy as jnp
from jax import lax
from jax.experimental import pallas as pl
from jax.experimental.pallas import tpu as pltpu
from jax.experimental.pallas import tpu_sc as plsc

MAX_SEQ = 8192
EMB_W = 1024

_NC = 2
_NS = 16
_NW = _NC * _NS

_B_PER_W = MAX_SEQ // _NW
_CHUNK = 32
_N_CHUNKS = _B_PER_W // _CHUNK
_NBUF = 2


def _gather_body(table_hbm, idx_hbm, out_hbm, idx_v, buf0, buf1, gsem, ssem):
    wid = lax.axis_index("s") * _NC + lax.axis_index("c")
    base = wid * _B_PER_W
    pltpu.sync_copy(idx_hbm.at[wid], idx_v)

    bufs = (buf0, buf1)
    pending = [None, None]
    for i in range(_N_CHUNKS):
        b = i % _NBUF
        if pending[b] is not None:
            pending[b].wait()
        g = pltpu.async_copy(table_hbm.at[idx_v.at[i]], bufs[b], gsem)
        g.wait()
        pending[b] = pltpu.async_copy(
            bufs[b], out_hbm.at[pl.ds(base + i * _CHUNK, _CHUNK)], ssem
        )
    for b in range(_NBUF):
        if pending[b] is not None:
            pending[b].wait()


@jax.jit
def _gather(inputs, embeddings):
    mesh = plsc.VectorSubcoreMesh(core_axis_name="c", subcore_axis_name="s")
    run = functools.partial(
        pl.kernel,
        mesh=mesh,
        out_type=jax.ShapeDtypeStruct((MAX_SEQ, EMB_W), jnp.float32),
        scratch_types=[
            pltpu.VMEM((_N_CHUNKS, _CHUNK), jnp.int32),
            pltpu.VMEM((_CHUNK, EMB_W), jnp.float32),
            pltpu.VMEM((_CHUNK, EMB_W), jnp.float32),
            pltpu.SemaphoreType.DMA,
            pltpu.SemaphoreType.DMA,
        ],
    )(_gather_body)
    return run(embeddings, inputs)


def kernel(inputs, embeddings):
    idx = inputs.astype(jnp.int32).reshape(_NW, _N_CHUNKS, _CHUNK)
    out = _gather(idx, embeddings)
    return jnp.expand_dims(out, 0)


# speedup vs baseline: 1.5744x; 1.0600x over previous
import functools

import jax
import jax.numpy as jnp
from jax import lax
from jax.experimental import pallas as pl
from jax.experimental.pallas import tpu as pltpu
from jax.experimental.pallas import tpu_sc as plsc

MAX_SEQ = 8192
EMB_W = 1024

_NC = 2
_NS = 16
_NW = _NC * _NS

_B_PER_W = MAX_SEQ // _NW
_CHUNK = 32
_N_CHUNKS = _B_PER_W // _CHUNK
_NBUF = 3


def _gather_body(table_hbm, idx_hbm, out_hbm, idx_v,
                 buf0, buf1, buf2, g0, g1, g2, s0, s1, s2):
    wid = lax.axis_index("s") * _NC + lax.axis_index("c")
    base = wid * _B_PER_W
    pltpu.sync_copy(idx_hbm.at[wid], idx_v)

    bufs = (buf0, buf1, buf2)
    gsems = (g0, g1, g2)
    ssems = (s0, s1, s2)
    gp = [None] * _NBUF
    sp = [None] * _NBUF
    for i in range(_NBUF):
        gp[i] = pltpu.async_copy(table_hbm.at[idx_v.at[i]], bufs[i], gsems[i])
    for i in range(_N_CHUNKS):
        b = i % _NBUF
        gp[b].wait()
        sp[b] = pltpu.async_copy(
            bufs[b], out_hbm.at[pl.ds(base + i * _CHUNK, _CHUNK)], ssems[b]
        )
        j = i + _NBUF
        if j < _N_CHUNKS:
            sp[b].wait()
            gp[b] = pltpu.async_copy(
                table_hbm.at[idx_v.at[j]], bufs[b], gsems[b]
            )
    for b in range(_NBUF):
        if sp[b] is not None:
            sp[b].wait()


@jax.jit
def _gather(inputs, embeddings):
    mesh = plsc.VectorSubcoreMesh(core_axis_name="c", subcore_axis_name="s")
    run = functools.partial(
        pl.kernel,
        mesh=mesh,
        out_type=jax.ShapeDtypeStruct((MAX_SEQ, EMB_W), jnp.float32),
        scratch_types=[
            pltpu.VMEM((_N_CHUNKS, _CHUNK), jnp.int32),
            pltpu.VMEM((_CHUNK, EMB_W), jnp.float32),
            pltpu.VMEM((_CHUNK, EMB_W), jnp.float32),
            pltpu.VMEM((_CHUNK, EMB_W), jnp.float32),
            pltpu.SemaphoreType.DMA,
            pltpu.SemaphoreType.DMA,
            pltpu.SemaphoreType.DMA,
            pltpu.SemaphoreType.DMA,
            pltpu.SemaphoreType.DMA,
            pltpu.SemaphoreType.DMA,
        ],
    )(_gather_body)
    return run(embeddings, inputs)


def kernel(inputs, embeddings):
    idx = inputs.astype(jnp.int32).reshape(_NW, _N_CHUNKS, _CHUNK)
    out = _gather(idx, embeddings)
    return jnp.expand_dims(out, 0)


# speedup vs baseline: 1.5794x; 1.0032x over previous
import functools

import jax
import jax.numpy as jnp
from jax import lax
from jax.experimental import pallas as pl
from jax.experimental.pallas import tpu as pltpu
from jax.experimental.pallas import tpu_sc as plsc

MAX_SEQ = 8192
EMB_W = 1024

_NC = 2
_NS = 16
_NW = _NC * _NS

_B_PER_W = MAX_SEQ // _NW
_CHUNK = 16
_N_CHUNKS = _B_PER_W // _CHUNK
_NBUF = 6


def _gather_body(table_hbm, idx_hbm, out_hbm, idx_v, *scratch):
    bufs = scratch[:_NBUF]
    gsems = scratch[_NBUF:2 * _NBUF]
    ssems = scratch[2 * _NBUF:3 * _NBUF]

    wid = lax.axis_index("s") * _NC + lax.axis_index("c")
    base = wid * _B_PER_W
    pltpu.sync_copy(idx_hbm.at[wid], idx_v)

    gp = [None] * _NBUF
    sp = [None] * _NBUF
    for i in range(min(_NBUF, _N_CHUNKS)):
        gp[i] = pltpu.async_copy(table_hbm.at[idx_v.at[i]], bufs[i], gsems[i])
    for i in range(_N_CHUNKS):
        b = i % _NBUF
        gp[b].wait()
        sp[b] = pltpu.async_copy(
            bufs[b], out_hbm.at[pl.ds(base + i * _CHUNK, _CHUNK)], ssems[b]
        )
        j = i + _NBUF
        if j < _N_CHUNKS:
            sp[b].wait()
            gp[b] = pltpu.async_copy(
                table_hbm.at[idx_v.at[j]], bufs[b], gsems[b]
            )
    for b in range(_NBUF):
        if sp[b] is not None:
            sp[b].wait()


@jax.jit
def _gather(inputs, embeddings):
    mesh = plsc.VectorSubcoreMesh(core_axis_name="c", subcore_axis_name="s")
    run = functools.partial(
        pl.kernel,
        mesh=mesh,
        out_type=jax.ShapeDtypeStruct((MAX_SEQ, EMB_W), jnp.float32),
        scratch_types=[pltpu.VMEM((_N_CHUNKS, _CHUNK), jnp.int32)]
        + [pltpu.VMEM((_CHUNK, EMB_W), jnp.float32) for _ in range(_NBUF)]
        + [pltpu.SemaphoreType.DMA for _ in range(2 * _NBUF)],
    )(_gather_body)
    return run(embeddings, inputs)


def kernel(inputs, embeddings):
    idx = inputs.astype(jnp.int32).reshape(_NW, _N_CHUNKS, _CHUNK)
    out = _gather(idx, embeddings)
    return jnp.expand_dims(out, 0)


# speedup vs baseline: 1.5970x; 1.0112x over previous
import functools

import jax
import jax.numpy as jnp
from jax import lax
from jax.experimental import pallas as pl
from jax.experimental.pallas import tpu as pltpu
from jax.experimental.pallas import tpu_sc as plsc

MAX_SEQ = 8192
EMB_W = 1024

_NC = 2
_NS = 16
_NW = _NC * _NS

_B_PER_W = MAX_SEQ // _NW
_CHUNK = 16
_N_CHUNKS = _B_PER_W // _CHUNK
_NBUF = 6


def _gather_body(table_hbm, idx_hbm, out_hbm, idx_v, *scratch):
    bufs = scratch[:_NBUF]
    gsems = scratch[_NBUF:2 * _NBUF]
    ssems = scratch[2 * _NBUF:3 * _NBUF]

    wid = lax.axis_index("s") * _NC + lax.axis_index("c")
    base = wid * _B_PER_W
    pltpu.sync_copy(idx_hbm.at[pl.ds(base, _B_PER_W)], idx_v)

    def chunk_idx(i):
        return idx_v[pl.ds(i * _CHUNK, _CHUNK)]

    gp = [None] * _NBUF
    sp = [None] * _NBUF
    for i in range(min(_NBUF, _N_CHUNKS)):
        gp[i] = pltpu.async_copy(table_hbm.at[chunk_idx(i)], bufs[i], gsems[i])
    for i in range(_N_CHUNKS):
        b = i % _NBUF
        gp[b].wait()
        sp[b] = pltpu.async_copy(
            bufs[b], out_hbm.at[pl.ds(base + i * _CHUNK, _CHUNK)], ssems[b]
        )
        j = i + _NBUF
        if j < _N_CHUNKS:
            sp[b].wait()
            gp[b] = pltpu.async_copy(
                table_hbm.at[chunk_idx(j)], bufs[b], gsems[b]
            )
    for b in range(_NBUF):
        if sp[b] is not None:
            sp[b].wait()


@jax.jit
def _gather(inputs, embeddings):
    mesh = plsc.VectorSubcoreMesh(core_axis_name="c", subcore_axis_name="s")
    run = functools.partial(
        pl.kernel,
        mesh=mesh,
        out_type=jax.ShapeDtypeStruct((MAX_SEQ, EMB_W), jnp.float32),
        scratch_types=[pltpu.VMEM((_B_PER_W,), jnp.int32)]
        + [pltpu.VMEM((_CHUNK, EMB_W), jnp.float32) for _ in range(_NBUF)]
        + [pltpu.SemaphoreType.DMA for _ in range(2 * _NBUF)],
    )(_gather_body)
    return run(embeddings, inputs)


def kernel(inputs, embeddings):
    out = _gather(inputs.astype(jnp.int32), embeddings)
    return jnp.expand_dims(out, 0)
